# baseline (speedup 1.0000x reference)
import functools

import jax
import jax.numpy as jnp
from jax import lax
from jax.experimental import pallas as pl
from jax.experimental.pallas import tpu as pltpu
from jax.experimental.pallas import tpu_sc as plsc

_UNITS = 26
_LOWER = -17.0
_UPPER = 8.0
_BW = 0.05
_NUM_BUCKETS = int((_UPPER - _LOWER) / _BW) + 1
_RESIDUE = _LOWER - _BW
_BATCH = 4096

_KPAD = 512
_CHUNKS = _KPAD // 16


def _sc_body(x_hbm, w_hbm, b_hbm, out_hbm,
             x_v, out_v, w_v, b_v, t1_v, t2_v, sem, sem_wb):
    u = lax.axis_index("s") * 2 + lax.axis_index("c")

    @pl.when(u < _UNITS)
    def _():
        base = u * _BATCH
        c1 = pltpu.async_copy(x_hbm.at[pl.ds(base, _BATCH)], x_v, sem)
        c2 = pltpu.async_copy(w_hbm.at[pl.ds(u * _KPAD, _KPAD)], w_v, sem_wb)
        c3 = pltpu.async_copy(b_hbm, b_v, sem_wb)
        c2.wait()
        c3.wait()

        bias_u = plsc.load_gather(b_v, [u + jnp.zeros((16,), jnp.int32)])

        def chunk(c, carry):
            v = jnp.maximum(w_v[pl.ds(c * 16, 16)], jnp.float32(0.0))
            incl = plsc.cumsum(v)
            t1_v[pl.ds(c * 16, 16)] = (
                (incl - v + carry) * jnp.float32(_BW)
                + jnp.float32(_RESIDUE) + bias_u
            )
            t2_v[pl.ds(c * 16, 16)] = v
            return carry + jnp.sum(v)

        lax.fori_loop(0, _CHUNKS, chunk, jnp.zeros((16,), jnp.float32))
        c1.wait()

        @plsc.parallel_loop(0, _BATCH, step=16, unroll=4)
        def _loop(off):
            xv = x_v[pl.ds(off, 16)]
            xc = jnp.clip(xv, jnp.float32(_LOWER + 1e-9), jnp.float32(_UPPER - 1e-9))
            t = (xc - jnp.float32(_LOWER) + jnp.float32(_BW)) * jnp.float32(1.0 / _BW)
            idx = jnp.clip(t.astype(jnp.int32), 0, _NUM_BUCKETS - 1)
            delta = (
                xc - jnp.float32(_LOWER) + jnp.float32(_BW)
                - idx.astype(jnp.float32) * jnp.float32(_BW)
            )
            g1 = plsc.load_gather(t1_v, [idx])
            g2 = plsc.load_gather(t2_v, [idx])
            z = g1 + delta * g2
            out_v[pl.ds(off, 16)] = (
                jnp.float32(1.0) / (jnp.float32(1.0) + jnp.exp(-z))
            )

        pltpu.sync_copy(out_v, out_hbm.at[pl.ds(base, _BATCH)])


def kernel(x, weights, bias):
    wp = jnp.pad(weights, ((0, 0), (0, _KPAD - _NUM_BUCKETS))).reshape(-1)
    bp = jnp.pad(bias, (0, 32 - _UNITS))
    xt = x.T.reshape(-1)
    mesh = plsc.VectorSubcoreMesh(core_axis_name="c", subcore_axis_name="s")
    run = functools.partial(
        pl.kernel,
        mesh=mesh,
        out_type=jax.ShapeDtypeStruct((_UNITS * _BATCH,), jnp.float32),
        scratch_types=[
            pltpu.VMEM((_BATCH,), jnp.float32),
            pltpu.VMEM((_BATCH,), jnp.float32),
            pltpu.VMEM((_KPAD,), jnp.float32),
            pltpu.VMEM((32,), jnp.float32),
            pltpu.VMEM((_KPAD,), jnp.float32),
            pltpu.VMEM((_KPAD,), jnp.float32),
            pltpu.SemaphoreType.DMA,
            pltpu.SemaphoreType.DMA,
        ],
        compiler_params=pltpu.CompilerParams(needs_layout_passes=False),
    )(_sc_body)
    out = run(xt, wp, bp)
    return out.reshape(_UNITS, _BATCH).T
